# baseline (speedup 1.0000x reference)
import functools

import jax
import jax.numpy as jnp
from jax import lax
from jax.experimental import pallas as pl
from jax.experimental.pallas import tpu as pltpu
from jax.experimental.pallas import tpu_sc as plsc

_K = 16
_NB = _K + 1
_RB = 256
_NW = 32
_L = 16


_BIGF = 3.0e38
_DEPTH = 6


def _packed_keys(x_ref, xt_ref):
    xb = x_ref[0]
    xt = xt_ref[0]
    dot = jnp.dot(xb, xt, preferred_element_type=jnp.float32)
    sqb = jnp.sum(xb * xb, axis=1, keepdims=True)
    sqa = jnp.sum(xt * xt, axis=0, keepdims=True)
    d2 = jnp.minimum(sqb + sqa - 2.0 * dot, 1e30)
    bits = lax.bitcast_convert_type(d2, jnp.int32)
    iota = lax.broadcasted_iota(jnp.int32, d2.shape, 1)
    return lax.bitcast_convert_type(
        ((bits & jnp.int32(-4096)) | iota) + jnp.int32(0x00800000),
        jnp.float32)


def _oem(a, b):
    if not a:
        return b
    if not b:
        return a
    if len(a) == 1 and len(b) == 1:
        return [jnp.minimum(a[0], b[0]), jnp.maximum(a[0], b[0])]
    ev = _oem(a[0::2], b[0::2])
    od = _oem(a[1::2], b[1::2])
    out = [ev[0]]
    ev = ev[1:]
    for i in range(max(len(od), len(ev))):
        if i < len(od) and i < len(ev):
            out.append(jnp.minimum(od[i], ev[i]))
            out.append(jnp.maximum(od[i], ev[i]))
        elif i < len(od):
            out.append(od[i])
        else:
            out.append(ev[i])
    return out


def _topk_body(x_ref, xt_ref, out_ref):
    keys = _packed_keys(x_ref, xt_ref)
    nseg = keys.shape[1] // 128
    lists = [[keys[:, c * 128:(c + 1) * 128]] for c in range(nseg)]
    while len(lists) > 1:
        nxt = [_oem(lists[i], lists[i + 1])[:_DEPTH]
               for i in range(0, len(lists) - 1, 2)]
        if len(lists) % 2:
            nxt.append(lists[-1])
        lists = nxt
    m = lists[0][:_DEPTH]
    kmins = []
    for t in range(_NB):
        kmin = jnp.min(m[0], axis=1, keepdims=True)
        kmins.append(kmin)
        if t < _NB - 1:
            eq = m[0] == kmin
            for lv in range(_DEPTH - 1):
                m[lv] = jnp.where(eq, m[lv + 1], m[lv])
            m[_DEPTH - 1] = jnp.where(eq, _BIGF, m[_DEPTH - 1])
    idx = lax.bitcast_convert_type(jnp.concatenate(kmins, axis=1),
                                   jnp.int32) & 0xFFF
    dirty = jnp.max(m[0], axis=1, keepdims=True) >= _BIGF
    out_ref[0] = jnp.where(dirty, -1, idx)


def _topk_body_slow(x_ref, xt_ref, out_ref):
    keys = _packed_keys(x_ref, xt_ref)
    kmins = []
    for _ in range(_NB):
        kmin = jnp.min(keys, axis=1, keepdims=True)
        kmins.append(kmin)
        keys = jnp.where(keys == kmin, _BIGF, keys)
    idx = lax.bitcast_convert_type(jnp.concatenate(kmins, axis=1),
                                   jnp.int32) & 0xFFF
    out_ref[0] = idx


def _topk_call(body, x, xt):
    B, N, _ = x.shape
    return pl.pallas_call(
        body,
        grid=(B, N // _RB),
        in_specs=[
            pl.BlockSpec((1, _RB, 3), lambda b, r: (b, r, 0)),
            pl.BlockSpec((1, 3, N), lambda b, r: (b, 0, 0)),
        ],
        out_specs=pl.BlockSpec((1, _RB, _NB), lambda b, r: (b, r, 0)),
        out_shape=jax.ShapeDtypeStruct((B, N, _NB), jnp.int32),
    )(x, xt)


def _topk_tc(x, xt):
    idx = _topk_call(_topk_body, x, xt)
    return lax.cond(jnp.min(idx) < 0,
                    lambda: _topk_call(_topk_body_slow, x, xt),
                    lambda: idx)


def _rsqrt(x):
    i = plsc.bitcast(x, jnp.int32)
    y = plsc.bitcast(jnp.int32(0x5F3759DF) - (i >> 1), jnp.float32)
    for _ in range(4):
        y = y * (1.5 - 0.5 * x * y * y)
    return y


def _make_sc_fused(B, N):
    chunk = (B * N) // _NW
    ngroups = chunk // _L
    mesh = plsc.VectorSubcoreMesh(core_axis_name="c", subcore_axis_name="s")

    @functools.partial(
        pl.kernel,
        out_type=(
            jax.ShapeDtypeStruct((_NW, _L), jnp.float32),
            jax.ShapeDtypeStruct((_NW, _L), jnp.float32),
        ),
        mesh=mesh,
        scratch_types=[
            pltpu.VMEM((3 * N,), jnp.float32),
            pltpu.VMEM((3 * N,), jnp.float32),
            pltpu.VMEM((_NB * chunk,), jnp.int32),
            pltpu.VMEM((9 * chunk,), jnp.float32),
            pltpu.VMEM((9 * N,), jnp.float32),
            pltpu.VMEM((_L,), jnp.float32),
            pltpu.VMEM((_L,), jnp.float32),
            pltpu.VMEM_SHARED((2, 9 * N), jnp.float32),
        ],
        compiler_params=pltpu.CompilerParams(needs_layout_passes=False),
    )
    def fused(x_hbm, v_hbm, idx_hbm, cont_hbm, mom_hbm,
              xv, vv, idxv, jv, jt, cv, mv, jshared):
        c = lax.axis_index("c")
        s = lax.axis_index("s")
        slot = s // 8
        b = c * 2 + slot
        base = (s % 8) * chunk
        wid = c * 16 + s
        pltpu.sync_copy(x_hbm.at[b], xv)
        pltpu.sync_copy(v_hbm.at[b], vv)
        pltpu.sync_copy(idx_hbm.at[b, pl.ds(base * _NB, chunk * _NB)], idxv)
        iota = lax.iota(jnp.int32, _L)

        def group2(g, cacc):
            lp17 = (iota + g * _L) * _NB
            i0 = plsc.load_gather(idxv, [lp17])
            xj0 = [plsc.load_gather(xv, [i0 + d * N]) for d in range(3)]
            i03 = i0 * 3
            vj0 = [plsc.load_gather(vv, [i03 + d]) for d in range(3)]
            zero = jnp.zeros((_L,), jnp.float32)
            cont = [zero, zero, zero]
            jac = [zero] * 9
            for m in range(1, _NB):
                im = plsc.load_gather(idxv, [lp17 + m])
                xj = [plsc.load_gather(xv, [im + d * N]) for d in range(3)]
                im3 = im * 3
                vj = [plsc.load_gather(vv, [im3 + d]) for d in range(3)]
                dx = [xj0[d] - xj[d] for d in range(3)]
                dv = [vj0[d] - vj[d] for d in range(3)]
                r2 = dx[0] * dx[0] + dx[1] * dx[1] + dx[2] * dx[2]
                w1 = _rsqrt(r2)
                w2 = 1.0 / r2
                for d in range(3):
                    cont[d] = cont[d] + dx[d] * dv[d] * w2
                a = [dv[d] * w1 for d in range(3)]
                for p in range(3):
                    for q in range(3):
                        jac[3 * p + q] = jac[3 * p + q] + a[p] * dx[q]
            for p in range(9):
                jv[pl.ds(p * chunk + g * _L, _L)] = jac[p] * (1.0 / _K)
            return cacc + jnp.abs(cont[0]) + jnp.abs(cont[1]) + jnp.abs(cont[2])

        cacc = lax.fori_loop(0, ngroups, group2, jnp.zeros((_L,), jnp.float32))
        cv[...] = cacc * (1.0 / _K)
        pltpu.sync_copy(cv, cont_hbm.at[wid])

        for p in range(9):
            pltpu.sync_copy(jv.at[pl.ds(p * chunk, chunk)],
                            jshared.at[slot, pl.ds(p * N + base, chunk)])
        plsc.subcore_barrier()
        pltpu.sync_copy(jshared.at[slot], jt)

        def group3(g, macc):
            lp = iota + g * _L
            lp17 = lp * _NB
            i0 = plsc.load_gather(idxv, [lp17])
            xj0 = [plsc.load_gather(xv, [i0 + d * N]) for d in range(3)]
            jj0 = [plsc.load_gather(jt, [i0 + p * N]) for p in range(9)]
            zero = jnp.zeros((_L,), jnp.float32)
            lap = [zero, zero, zero]
            for m in range(1, _NB):
                im = plsc.load_gather(idxv, [lp17 + m])
                xj = [plsc.load_gather(xv, [im + d * N]) for d in range(3)]
                dx = [xj0[d] - xj[d] for d in range(3)]
                r2 = dx[0] * dx[0] + dx[1] * dx[1] + dx[2] * dx[2]
                w1 = _rsqrt(r2)
                jj = [plsc.load_gather(jt, [im + p * N]) for p in range(9)]
                for p in range(3):
                    acc = zero
                    for q in range(3):
                        acc = acc + (jj0[3 * p + q] - jj[3 * p + q]) * dx[q]
                    lap[p] = lap[p] + acc * w1
            ji = [jv[pl.ds(p * chunk + g * _L, _L)] for p in range(9)]
            lp3 = lp * 3
            vi = [plsc.load_gather(vv, [lp3 + d + base * 3]) for d in range(3)]
            mom = []
            for p in range(3):
                mp = ji[3 * p] * vi[0] + ji[3 * p + 1] * vi[1] + ji[3 * p + 2] * vi[2]
                mom.append(mp - lap[p] * (1.0 / _K))
            m2 = mom[0] * mom[0] + mom[1] * mom[1] + mom[2] * mom[2]
            norm = m2 * _rsqrt(jnp.maximum(m2, 1e-30))
            return macc + norm

        macc = lax.fori_loop(0, ngroups, group3, jnp.zeros((_L,), jnp.float32))
        mv[...] = macc
        pltpu.sync_copy(mv, mom_hbm.at[wid])

    return fused


def kernel(target_tensor, pred_tensor):
    xyz, vel = target_tensor, pred_tensor
    B, N, _ = xyz.shape
    xt = jnp.transpose(xyz, (0, 2, 1))
    idx = _topk_tc(xyz, xt)
    xtf = xt.reshape(B, 3 * N)
    vf = vel.reshape(B, N * 3)
    idxf = idx.reshape(B, N * _NB)
    cont_part, mom_part = _make_sc_fused(B, N)(xtf, vf, idxf)
    cont_loss = jnp.sum(cont_part) / (B * N * 3)
    mom_loss = jnp.sum(mom_part) / (B * N)
    return 0.5 * cont_loss + 0.5 * mom_loss


# speedup vs baseline: 1.1004x; 1.1004x over previous
import functools

import jax
import jax.numpy as jnp
from jax import lax
from jax.experimental import pallas as pl
from jax.experimental.pallas import tpu as pltpu
from jax.experimental.pallas import tpu_sc as plsc

_K = 16
_NB = _K + 1
_RB = 512
_NW = 32
_L = 16


_BIGF = 3.0e38
_DEPTH = 6


def _packed_keys(x_ref, xt_ref):
    xb = x_ref[0]
    xt = xt_ref[0]
    dot = jnp.dot(xb, xt, preferred_element_type=jnp.float32)
    sqb = jnp.sum(xb * xb, axis=1, keepdims=True)
    sqa = jnp.sum(xt * xt, axis=0, keepdims=True)
    d2 = jnp.minimum(sqb + sqa - 2.0 * dot, 1e30)
    bits = lax.bitcast_convert_type(d2, jnp.int32)
    iota = lax.broadcasted_iota(jnp.int32, d2.shape, 1)
    return lax.bitcast_convert_type(
        ((bits & jnp.int32(-4096)) | iota) + jnp.int32(0x00800000),
        jnp.float32)


def _oem(a, b):
    if not a:
        return b
    if not b:
        return a
    if len(a) == 1 and len(b) == 1:
        return [jnp.minimum(a[0], b[0]), jnp.maximum(a[0], b[0])]
    ev = _oem(a[0::2], b[0::2])
    od = _oem(a[1::2], b[1::2])
    out = [ev[0]]
    ev = ev[1:]
    for i in range(max(len(od), len(ev))):
        if i < len(od) and i < len(ev):
            out.append(jnp.minimum(od[i], ev[i]))
            out.append(jnp.maximum(od[i], ev[i]))
        elif i < len(od):
            out.append(od[i])
        else:
            out.append(ev[i])
    return out


def _topk_body(x_ref, xt_ref, out_ref):
    keys = _packed_keys(x_ref, xt_ref)
    nseg = keys.shape[1] // 128
    lists = [[keys[:, c * 128:(c + 1) * 128]] for c in range(nseg)]
    while len(lists) > 1:
        nxt = [_oem(lists[i], lists[i + 1])[:_DEPTH]
               for i in range(0, len(lists) - 1, 2)]
        if len(lists) % 2:
            nxt.append(lists[-1])
        lists = nxt
    m = lists[0][:_DEPTH]
    kmins = []
    for t in range(_NB):
        kmin = jnp.min(m[0], axis=1, keepdims=True)
        kmins.append(kmin)
        if t < _NB - 1:
            eq = m[0] == kmin
            for lv in range(_DEPTH - 1):
                m[lv] = jnp.where(eq, m[lv + 1], m[lv])
            m[_DEPTH - 1] = jnp.where(eq, _BIGF, m[_DEPTH - 1])
    idx = lax.bitcast_convert_type(jnp.concatenate(kmins, axis=1),
                                   jnp.int32) & 0xFFF
    dirty = jnp.max(m[0], axis=1, keepdims=True) >= _BIGF
    out_ref[0] = jnp.where(dirty, -1, idx)


def _topk_body_slow(x_ref, xt_ref, out_ref):
    keys = _packed_keys(x_ref, xt_ref)
    kmins = []
    for _ in range(_NB):
        kmin = jnp.min(keys, axis=1, keepdims=True)
        kmins.append(kmin)
        keys = jnp.where(keys == kmin, _BIGF, keys)
    idx = lax.bitcast_convert_type(jnp.concatenate(kmins, axis=1),
                                   jnp.int32) & 0xFFF
    out_ref[0] = idx


def _topk_call(body, x, xt):
    B, N, _ = x.shape
    return pl.pallas_call(
        body,
        grid=(B, N // _RB),
        in_specs=[
            pl.BlockSpec((1, _RB, 3), lambda b, r: (b, r, 0)),
            pl.BlockSpec((1, 3, N), lambda b, r: (b, 0, 0)),
        ],
        out_specs=pl.BlockSpec((1, _RB, _NB), lambda b, r: (b, r, 0)),
        out_shape=jax.ShapeDtypeStruct((B, N, _NB), jnp.int32),
    )(x, xt)


def _topk_tc(x, xt):
    idx = _topk_call(_topk_body, x, xt)
    return lax.cond(jnp.min(idx) < 0,
                    lambda: _topk_call(_topk_body_slow, x, xt),
                    lambda: idx)


def _rsqrt(x):
    i = plsc.bitcast(x, jnp.int32)
    y = plsc.bitcast(jnp.int32(0x5F3759DF) - (i >> 1), jnp.float32)
    for _ in range(4):
        y = y * (1.5 - 0.5 * x * y * y)
    return y


def _make_sc_fused(B, N):
    chunk = (B * N) // _NW
    ngroups = chunk // _L
    mesh = plsc.VectorSubcoreMesh(core_axis_name="c", subcore_axis_name="s")

    @functools.partial(
        pl.kernel,
        out_type=(
            jax.ShapeDtypeStruct((_NW, _L), jnp.float32),
            jax.ShapeDtypeStruct((_NW, _L), jnp.float32),
        ),
        mesh=mesh,
        scratch_types=[
            pltpu.VMEM((3 * N,), jnp.float32),
            pltpu.VMEM((3 * N,), jnp.float32),
            pltpu.VMEM((_NB * chunk,), jnp.int32),
            pltpu.VMEM((9 * chunk,), jnp.float32),
            pltpu.VMEM((9 * N,), jnp.float32),
            pltpu.VMEM((_L,), jnp.float32),
            pltpu.VMEM((_L,), jnp.float32),
            pltpu.VMEM_SHARED((2, 9 * N), jnp.float32),
        ],
        compiler_params=pltpu.CompilerParams(needs_layout_passes=False),
    )
    def fused(x_hbm, v_hbm, idx_hbm, cont_hbm, mom_hbm,
              xv, vv, idxv, jv, jt, cv, mv, jshared):
        c = lax.axis_index("c")
        s = lax.axis_index("s")
        slot = s // 8
        b = c * 2 + slot
        base = (s % 8) * chunk
        wid = c * 16 + s
        pltpu.sync_copy(x_hbm.at[b], xv)
        pltpu.sync_copy(v_hbm.at[b], vv)
        pltpu.sync_copy(idx_hbm.at[b, pl.ds(base * _NB, chunk * _NB)], idxv)
        iota = lax.iota(jnp.int32, _L)

        def group2(g, cacc):
            lp17 = (iota + g * _L) * _NB
            i0 = plsc.load_gather(idxv, [lp17])
            xj0 = [plsc.load_gather(xv, [i0 + d * N]) for d in range(3)]
            i03 = i0 * 3
            vj0 = [plsc.load_gather(vv, [i03 + d]) for d in range(3)]
            zero = jnp.zeros((_L,), jnp.float32)
            cont = [zero, zero, zero]
            jac = [zero] * 9
            for m in range(1, _NB):
                im = plsc.load_gather(idxv, [lp17 + m])
                xj = [plsc.load_gather(xv, [im + d * N]) for d in range(3)]
                im3 = im * 3
                vj = [plsc.load_gather(vv, [im3 + d]) for d in range(3)]
                dx = [xj0[d] - xj[d] for d in range(3)]
                dv = [vj0[d] - vj[d] for d in range(3)]
                r2 = dx[0] * dx[0] + dx[1] * dx[1] + dx[2] * dx[2]
                w1 = _rsqrt(r2)
                w2 = 1.0 / r2
                for d in range(3):
                    cont[d] = cont[d] + dx[d] * dv[d] * w2
                a = [dv[d] * w1 for d in range(3)]
                for p in range(3):
                    for q in range(3):
                        jac[3 * p + q] = jac[3 * p + q] + a[p] * dx[q]
            for p in range(9):
                jv[pl.ds(p * chunk + g * _L, _L)] = jac[p] * (1.0 / _K)
            return cacc + jnp.abs(cont[0]) + jnp.abs(cont[1]) + jnp.abs(cont[2])

        cacc = lax.fori_loop(0, ngroups, group2, jnp.zeros((_L,), jnp.float32))
        cv[...] = cacc * (1.0 / _K)
        pltpu.sync_copy(cv, cont_hbm.at[wid])

        for p in range(9):
            pltpu.sync_copy(jv.at[pl.ds(p * chunk, chunk)],
                            jshared.at[slot, pl.ds(p * N + base, chunk)])
        plsc.subcore_barrier()
        pltpu.sync_copy(jshared.at[slot], jt)

        def group3(g, macc):
            lp = iota + g * _L
            lp17 = lp * _NB
            i0 = plsc.load_gather(idxv, [lp17])
            xj0 = [plsc.load_gather(xv, [i0 + d * N]) for d in range(3)]
            jj0 = [plsc.load_gather(jt, [i0 + p * N]) for p in range(9)]
            zero = jnp.zeros((_L,), jnp.float32)
            lap = [zero, zero, zero]
            for m in range(1, _NB):
                im = plsc.load_gather(idxv, [lp17 + m])
                xj = [plsc.load_gather(xv, [im + d * N]) for d in range(3)]
                dx = [xj0[d] - xj[d] for d in range(3)]
                r2 = dx[0] * dx[0] + dx[1] * dx[1] + dx[2] * dx[2]
                w1 = _rsqrt(r2)
                jj = [plsc.load_gather(jt, [im + p * N]) for p in range(9)]
                for p in range(3):
                    acc = zero
                    for q in range(3):
                        acc = acc + (jj0[3 * p + q] - jj[3 * p + q]) * dx[q]
                    lap[p] = lap[p] + acc * w1
            ji = [jv[pl.ds(p * chunk + g * _L, _L)] for p in range(9)]
            lp3 = lp * 3
            vi = [plsc.load_gather(vv, [lp3 + d + base * 3]) for d in range(3)]
            mom = []
            for p in range(3):
                mp = ji[3 * p] * vi[0] + ji[3 * p + 1] * vi[1] + ji[3 * p + 2] * vi[2]
                mom.append(mp - lap[p] * (1.0 / _K))
            m2 = mom[0] * mom[0] + mom[1] * mom[1] + mom[2] * mom[2]
            norm = m2 * _rsqrt(jnp.maximum(m2, 1e-30))
            return macc + norm

        macc = lax.fori_loop(0, ngroups, group3, jnp.zeros((_L,), jnp.float32))
        mv[...] = macc
        pltpu.sync_copy(mv, mom_hbm.at[wid])

    return fused


def kernel(target_tensor, pred_tensor):
    xyz, vel = target_tensor, pred_tensor
    B, N, _ = xyz.shape
    xt = jnp.transpose(xyz, (0, 2, 1))
    idx = _topk_tc(xyz, xt)
    xtf = xt.reshape(B, 3 * N)
    vf = vel.reshape(B, N * 3)
    idxf = idx.reshape(B, N * _NB)
    cont_part, mom_part = _make_sc_fused(B, N)(xtf, vf, idxf)
    cont_loss = jnp.sum(cont_part) / (B * N * 3)
    mom_loss = jnp.sum(mom_part) / (B * N)
    return 0.5 * cont_loss + 0.5 * mom_loss
